# baseline (speedup 1.0000x reference)
import functools

import jax
import jax.numpy as jnp
from jax import lax
from jax.experimental import pallas as pl
from jax.experimental.pallas import tpu as pltpu
from jax.experimental.pallas import tpu_sc as plsc

N = 10000
D = 256
DH = 128
H2 = 512
E = 160000
B = 128
NS = 16
CHUNK = 80
CPT = (E // NS) // CHUNK
NB = 5
CPB = CPT // NB
ZR = 624
BLK = 2000
G = N // BLK


def _sc_scatter(h0, h1, src_r, dst_r):
    mesh = plsc.VectorSubcoreMesh(core_axis_name="c", subcore_axis_name="s")

    @functools.partial(
        pl.kernel,
        out_type=(jax.ShapeDtypeStruct((N, DH), jnp.float32),
                  jax.ShapeDtypeStruct((N, DH), jnp.float32)),
        mesh=mesh,
        scratch_types=[
            pltpu.VMEM((CPB, CHUNK), jnp.int32),
            pltpu.VMEM((CPB, CHUNK), jnp.int32),
            pltpu.VMEM((CHUNK, DH), jnp.float32),
            pltpu.VMEM((CHUNK, DH), jnp.float32),
            pltpu.VMEM((CHUNK, DH), jnp.float32),
            pltpu.VMEM_SHARED((N, DH), jnp.float32),
            pltpu.SemaphoreType.DMA,
            pltpu.SemaphoreType.DMA,
        ],
    )
    def k(h0_hbm, h1_hbm, src_hbm, dst_hbm, a0_hbm, a1_hbm,
          srcb, dstb, rows_a, rows_b, rows_c, acc, gsem, ssem):
        c = lax.axis_index("c")
        s = lax.axis_index("s")

        def part_copy(src, dst):
            @pl.when(s < NS - 1)
            def _():
                pltpu.sync_copy(src.at[pl.ds(s * ZR, ZR)],
                                dst.at[pl.ds(s * ZR, ZR)])

            @pl.when(s == NS - 1)
            def _():
                pltpu.sync_copy(src.at[pl.ds((NS - 1) * ZR, N - (NS - 1) * ZR)],
                                dst.at[pl.ds((NS - 1) * ZR, N - (NS - 1) * ZR)])

        @pl.when(c == 0)
        def _():
            part_copy(h0_hbm, acc)

        @pl.when(c == 1)
        def _():
            part_copy(h1_hbm, acc)

        plsc.subcore_barrier()

        bufs = (rows_a, rows_b, rows_c)

        def run(h_hbm):
            def fire_g(j, buf):
                pltpu.async_copy(h_hbm.at[srcb.at[j]], buf, gsem)

            def fire_s(j, buf):
                pltpu.async_copy(buf, acc.at[dstb.at[j]], ssem, add=True)

            def drain(buf, sem):
                pltpu.make_async_copy(h_hbm.at[srcb.at[0]], buf, sem).wait()

            fire_g(0, bufs[0])
            fire_g(1, bufs[1])
            fire_g(2, bufs[2])
            drain(bufs[0], gsem)
            fire_s(0, bufs[0])

            def slot(j, b, b_next):
                drain(b_next, ssem)
                fire_g(j + 2, b_next)
                drain(b, gsem)
                fire_s(j, b)

            def step(k, carry):
                slot(3 * k + 1, bufs[1], bufs[0])
                slot(3 * k + 2, bufs[2], bufs[1])
                slot(3 * k + 3, bufs[0], bufs[2])
                return carry
            lax.fori_loop(0, (CPB - 4) // 3, step, 0, unroll=False)
            slot(CPB - 3, bufs[(CPB - 3) % 3], bufs[(CPB - 1) % 3])
            drain(bufs[(CPB - 2) % 3], gsem)
            fire_s(CPB - 2, bufs[(CPB - 2) % 3])
            drain(bufs[(CPB - 1) % 3], gsem)
            fire_s(CPB - 1, bufs[(CPB - 1) % 3])
            drain(bufs[0], ssem)
            drain(bufs[1], ssem)
            drain(bufs[2], ssem)

        for b in range(NB):
            pltpu.sync_copy(src_hbm.at[s, b], srcb)
            pltpu.sync_copy(dst_hbm.at[s, b], dstb)

            @pl.when(c == 0)
            def _():
                run(h0_hbm)

            @pl.when(c == 1)
            def _():
                run(h1_hbm)

        plsc.subcore_barrier()

        @pl.when(c == 0)
        def _():
            part_copy(acc, a0_hbm)

        @pl.when(c == 1)
        def _():
            part_copy(acc, a1_hbm)

    return k(h0, h1, src_r, dst_r)


def _stats_update(st, x, j, g_last):
    @pl.when(j == 0)
    def _():
        st[...] = jnp.zeros_like(st)

    st[0:1, :] = st[0:1, :] + jnp.sum(x, axis=0, keepdims=True)
    st[1:2, :] = st[1:2, :] + jnp.sum(x * x, axis=0, keepdims=True)

    @pl.when(j == g_last)
    def _():
        m = st[0:1, :] / N
        v = st[1:2, :] / N - m * m
        st[0:1, :] = m
        st[1:2, :] = lax.rsqrt(v + 1e-5)


def _dense_phases01(a0_r, a1_r, w1_r, b1_r, g1_r, be1_r, w2_r, b2_r,
                    t_s, u_s, st1, st2, p, j):
    @pl.when(p == 0)
    def _():
        s = jnp.concatenate([a0_r[...], a1_r[...]], axis=1)
        t = jnp.dot(s, w1_r[...], preferred_element_type=jnp.float32,
                    precision=lax.Precision.HIGHEST)
        t = t + b1_r[...]
        t_s[pl.ds(j * BLK, BLK), :] = t
        _stats_update(st1, t, j, G - 1)

    @pl.when(p == 1)
    def _():
        t = t_s[pl.ds(j * BLK, BLK), :]
        xb = (t - st1[0:1, :]) * st1[1:2, :] * g1_r[...] + be1_r[...]
        xb = jnp.maximum(xb, 0.0)
        u = jnp.dot(xb, w2_r[...], preferred_element_type=jnp.float32,
                    precision=lax.Precision.HIGHEST)
        u = u + b2_r[...]
        u_s[pl.ds(j * BLK, BLK), :] = u
        _stats_update(st2, u, j, G - 1)


def _layer_body(a0_r, a1_r, w1_r, b1_r, g1_r, be1_r, w2_r, b2_r, g2_r, be2_r,
                h0_r, h1_r, t_s, u_s, st1, st2):
    p = pl.program_id(0)
    j = pl.program_id(1)
    _dense_phases01(a0_r, a1_r, w1_r, b1_r, g1_r, be1_r, w2_r, b2_r,
                    t_s, u_s, st1, st2, p, j)

    @pl.when(p == 2)
    def _():
        u = u_s[pl.ds(j * BLK, BLK), :]
        h = (u - st2[0:1, :]) * st2[1:2, :] * g2_r[...] + be2_r[...]
        h = jnp.maximum(h, 0.0)
        h0_r[...] = h[:, :DH]
        h1_r[...] = h[:, DH:]


def _readout_body(a0_r, a1_r, w1_r, b1_r, g1_r, be1_r, w2_r, b2_r, g2_r,
                  be2_r, bt_r, wc_r, bc_r, o_r, t_s, u_s, st1, st2, sums,
                  cnts):
    p = pl.program_id(0)
    j = pl.program_id(1)
    _dense_phases01(a0_r, a1_r, w1_r, b1_r, g1_r, be1_r, w2_r, b2_r,
                    t_s, u_s, st1, st2, p, j)

    @pl.when(p == 2)
    def _():
        u = u_s[pl.ds(j * BLK, BLK), :]
        post = (u - st2[0:1, :]) * st2[1:2, :] * g2_r[...] + be2_r[...]
        bb = bt_r[...]
        ids = lax.broadcasted_iota(jnp.int32, (BLK, B), 1)
        oh = (bb == ids).astype(jnp.float32)
        dn = (((0,), (0,)), ((), ()))
        seg = lax.dot_general(oh, post, dn, preferred_element_type=jnp.float32)
        cnt = lax.dot_general(oh, jnp.ones((BLK, 128), jnp.float32), dn,
                              preferred_element_type=jnp.float32)

        @pl.when(j == 0)
        def _():
            sums[...] = jnp.zeros_like(sums)
            cnts[...] = jnp.zeros_like(cnts)

        sums[...] = sums[...] + seg
        cnts[...] = cnts[...] + cnt

        @pl.when(j == G - 1)
        def _():
            ro = sums[...] / jnp.maximum(cnts[:, 0:1], 1.0)
            o_r[...] = jnp.dot(ro, wc_r[...],
                               preferred_element_type=jnp.float32) + bc_r[...]


def _gated(block, active_phase):
    def ix(p, j):
        return (jnp.where(p == active_phase, j, 0), 0)
    return pl.BlockSpec(block, ix)


def _const(block):
    return pl.BlockSpec(block, lambda p, j: (0, 0))


_PARAM_SPECS = [
    _const((D, H2)),
    _const((1, H2)),
    _const((1, H2)),
    _const((1, H2)),
    _const((H2, D)),
    _const((1, D)),
    _const((1, D)),
    _const((1, D)),
]

_DENSE_SCRATCH = [
    pltpu.VMEM((N, H2), jnp.float32),
    pltpu.VMEM((N, D), jnp.float32),
    pltpu.VMEM((2, H2), jnp.float32),
    pltpu.VMEM((2, D), jnp.float32),
]


def _dense_layer(a0, a1, params):
    return pl.pallas_call(
        _layer_body,
        grid=(3, G),
        in_specs=[_gated((BLK, DH), 0), _gated((BLK, DH), 0)] + _PARAM_SPECS,
        out_specs=(_gated((BLK, DH), 2), _gated((BLK, DH), 2)),
        out_shape=(jax.ShapeDtypeStruct((N, DH), jnp.float32),
                   jax.ShapeDtypeStruct((N, DH), jnp.float32)),
        scratch_shapes=_DENSE_SCRATCH,
        compiler_params=pltpu.CompilerParams(
            dimension_semantics=("arbitrary", "arbitrary")),
    )(a0, a1, *params)


def _dense_readout(a0, a1, params, batch_col, wc, bc):
    return pl.pallas_call(
        _readout_body,
        grid=(3, G),
        in_specs=[_gated((BLK, DH), 0), _gated((BLK, DH), 0)] + _PARAM_SPECS
        + [_gated((BLK, 1), 2), _const((D, 1)), _const((1, 1))],
        out_specs=pl.BlockSpec((B, 1), lambda p, j: (0, 0)),
        out_shape=jax.ShapeDtypeStruct((B, 1), jnp.float32),
        scratch_shapes=_DENSE_SCRATCH + [pltpu.VMEM((B, D), jnp.float32),
                                         pltpu.VMEM((B, 128), jnp.float32)],
        compiler_params=pltpu.CompilerParams(
            dimension_semantics=("arbitrary", "arbitrary")),
    )(a0, a1, *params, batch_col, wc, bc)


def kernel(x, edge_index, batch, batch_size, W1, b1, g1, be1, W2, b2, g2, be2,
           Wc, bc):
    src_r = edge_index[0].reshape(NS, NB, CPB, CHUNK)
    dst_r = edge_index[1].reshape(NS, NB, CPB, CHUNK)
    batch_col = batch.reshape(N, 1)
    h0 = x[:, :DH]
    h1 = x[:, DH:]
    out = None
    for i in range(3):
        params = (W1[i], b1[i].reshape(1, H2), g1[i].reshape(1, H2),
                  be1[i].reshape(1, H2), W2[i], b2[i].reshape(1, D),
                  g2[i].reshape(1, D), be2[i].reshape(1, D))
        a0, a1 = _sc_scatter(h0, h1, src_r, dst_r)
        if i < 2:
            h0, h1 = _dense_layer(a0, a1, params)
        else:
            out = _dense_readout(a0, a1, params, batch_col, Wc,
                                 bc.reshape(1, 1))
    return out


# speedup vs baseline: 1.2152x; 1.2152x over previous
import functools

import jax
import jax.numpy as jnp
from jax import lax
from jax.experimental import pallas as pl
from jax.experimental.pallas import tpu as pltpu
from jax.experimental.pallas import tpu_sc as plsc

N = 10000
D = 256
DH = 128
H2 = 512
E = 160000
B = 128
NS = 16
CHUNK = 80
CPT = (E // NS) // CHUNK
NB = 5
CPB = CPT // NB
ZR = 624
BLK = 2000
G = N // BLK


def _sc_scatter(h0, h1, src_r, dst_r):
    mesh = plsc.VectorSubcoreMesh(core_axis_name="c", subcore_axis_name="s")

    @functools.partial(
        pl.kernel,
        out_type=(jax.ShapeDtypeStruct((N, DH), jnp.float32),
                  jax.ShapeDtypeStruct((N, DH), jnp.float32)),
        mesh=mesh,
        scratch_types=[
            pltpu.VMEM((CPB, CHUNK), jnp.int32),
            pltpu.VMEM((CPB, CHUNK), jnp.int32),
            pltpu.VMEM((CHUNK, DH), jnp.float32),
            pltpu.VMEM((CHUNK, DH), jnp.float32),
            pltpu.VMEM((CHUNK, DH), jnp.float32),
            pltpu.VMEM_SHARED((N, DH), jnp.float32),
            pltpu.SemaphoreType.DMA,
            pltpu.SemaphoreType.DMA,
        ],
    )
    def k(h0_hbm, h1_hbm, src_hbm, dst_hbm, a0_hbm, a1_hbm,
          srcb, dstb, rows_a, rows_b, rows_c, acc, gsem, ssem):
        c = lax.axis_index("c")
        s = lax.axis_index("s")

        def part_copy(src, dst):
            @pl.when(s < NS - 1)
            def _():
                pltpu.sync_copy(src.at[pl.ds(s * ZR, ZR)],
                                dst.at[pl.ds(s * ZR, ZR)])

            @pl.when(s == NS - 1)
            def _():
                pltpu.sync_copy(src.at[pl.ds((NS - 1) * ZR, N - (NS - 1) * ZR)],
                                dst.at[pl.ds((NS - 1) * ZR, N - (NS - 1) * ZR)])

        @pl.when(c == 0)
        def _():
            part_copy(h0_hbm, acc)

        @pl.when(c == 1)
        def _():
            part_copy(h1_hbm, acc)

        plsc.subcore_barrier()

        bufs = (rows_a, rows_b, rows_c)

        def run(h_hbm):
            def fire_g(j, buf):
                pltpu.async_copy(h_hbm.at[srcb.at[j]], buf, gsem)

            def fire_s(j, buf):
                pltpu.async_copy(buf, acc.at[dstb.at[j]], ssem, add=True)

            def drain(buf, sem):
                pltpu.make_async_copy(h_hbm.at[srcb.at[0]], buf, sem).wait()

            fire_g(0, bufs[0])
            fire_g(1, bufs[1])
            fire_g(2, bufs[2])
            drain(bufs[0], gsem)
            fire_s(0, bufs[0])

            def slot(j, b, b_next):
                drain(b_next, ssem)
                fire_g(j + 2, b_next)
                drain(b, gsem)
                fire_s(j, b)

            def step(k, carry):
                slot(3 * k + 1, bufs[1], bufs[0])
                slot(3 * k + 2, bufs[2], bufs[1])
                slot(3 * k + 3, bufs[0], bufs[2])
                return carry
            lax.fori_loop(0, (CPB - 4) // 3, step, 0, unroll=False)
            slot(CPB - 3, bufs[(CPB - 3) % 3], bufs[(CPB - 1) % 3])
            drain(bufs[(CPB - 2) % 3], gsem)
            fire_s(CPB - 2, bufs[(CPB - 2) % 3])
            drain(bufs[(CPB - 1) % 3], gsem)
            fire_s(CPB - 1, bufs[(CPB - 1) % 3])
            drain(bufs[0], ssem)
            drain(bufs[1], ssem)
            drain(bufs[2], ssem)

        for b in range(NB):
            pltpu.sync_copy(src_hbm.at[s, b], srcb)
            pltpu.sync_copy(dst_hbm.at[s, b], dstb)

            @pl.when(c == 0)
            def _():
                run(h0_hbm)

            @pl.when(c == 1)
            def _():
                run(h1_hbm)

        plsc.subcore_barrier()

        @pl.when(c == 0)
        def _():
            part_copy(acc, a0_hbm)

        @pl.when(c == 1)
        def _():
            part_copy(acc, a1_hbm)

    return k(h0, h1, src_r, dst_r)


def _stats_update(st, x, j, g_last):
    @pl.when(j == 0)
    def _():
        st[...] = jnp.zeros_like(st)

    st[0:1, :] = st[0:1, :] + jnp.sum(x, axis=0, keepdims=True)
    st[1:2, :] = st[1:2, :] + jnp.sum(x * x, axis=0, keepdims=True)

    @pl.when(j == g_last)
    def _():
        m = st[0:1, :] / N
        v = st[1:2, :] / N - m * m
        st[0:1, :] = m
        st[1:2, :] = lax.rsqrt(v + 1e-5)


def _dense_phases01(a0_r, a1_r, w1_r, b1_r, g1_r, be1_r, w2_r, b2_r,
                    t_s, u_s, st1, st2, p, j):
    @pl.when(p == 0)
    def _():
        s = jnp.concatenate([a0_r[...], a1_r[...]], axis=1)
        t = jnp.dot(s, w1_r[...], preferred_element_type=jnp.float32)
        t = t + b1_r[...]
        t_s[pl.ds(j * BLK, BLK), :] = t
        _stats_update(st1, t, j, G - 1)

    @pl.when(p == 1)
    def _():
        t = t_s[pl.ds(j * BLK, BLK), :]
        xb = (t - st1[0:1, :]) * st1[1:2, :] * g1_r[...] + be1_r[...]
        xb = jnp.maximum(xb, 0.0)
        u = jnp.dot(xb, w2_r[...], preferred_element_type=jnp.float32)
        u = u + b2_r[...]
        u_s[pl.ds(j * BLK, BLK), :] = u
        _stats_update(st2, u, j, G - 1)


def _layer_body(a0_r, a1_r, w1_r, b1_r, g1_r, be1_r, w2_r, b2_r, g2_r, be2_r,
                h0_r, h1_r, t_s, u_s, st1, st2):
    p = pl.program_id(0)
    j = pl.program_id(1)
    _dense_phases01(a0_r, a1_r, w1_r, b1_r, g1_r, be1_r, w2_r, b2_r,
                    t_s, u_s, st1, st2, p, j)

    @pl.when(p == 2)
    def _():
        u = u_s[pl.ds(j * BLK, BLK), :]
        h = (u - st2[0:1, :]) * st2[1:2, :] * g2_r[...] + be2_r[...]
        h = jnp.maximum(h, 0.0)
        h0_r[...] = h[:, :DH]
        h1_r[...] = h[:, DH:]


def _readout_body(a0_r, a1_r, w1_r, b1_r, g1_r, be1_r, w2_r, b2_r, g2_r,
                  be2_r, bt_r, wc_r, bc_r, o_r, t_s, u_s, st1, st2, sums,
                  cnts):
    p = pl.program_id(0)
    j = pl.program_id(1)
    _dense_phases01(a0_r, a1_r, w1_r, b1_r, g1_r, be1_r, w2_r, b2_r,
                    t_s, u_s, st1, st2, p, j)

    @pl.when(p == 2)
    def _():
        u = u_s[pl.ds(j * BLK, BLK), :]
        post = (u - st2[0:1, :]) * st2[1:2, :] * g2_r[...] + be2_r[...]
        bb = bt_r[...]
        ids = lax.broadcasted_iota(jnp.int32, (BLK, B), 1)
        oh = (bb == ids).astype(jnp.float32)
        dn = (((0,), (0,)), ((), ()))
        seg = lax.dot_general(oh, post, dn, preferred_element_type=jnp.float32)
        cnt = lax.dot_general(oh, jnp.ones((BLK, 128), jnp.float32), dn,
                              preferred_element_type=jnp.float32)

        @pl.when(j == 0)
        def _():
            sums[...] = jnp.zeros_like(sums)
            cnts[...] = jnp.zeros_like(cnts)

        sums[...] = sums[...] + seg
        cnts[...] = cnts[...] + cnt

        @pl.when(j == G - 1)
        def _():
            ro = sums[...] / jnp.maximum(cnts[:, 0:1], 1.0)
            o_r[...] = jnp.dot(ro, wc_r[...],
                               preferred_element_type=jnp.float32) + bc_r[...]


def _gated(block, active_phase):
    def ix(p, j):
        return (jnp.where(p == active_phase, j, 0), 0)
    return pl.BlockSpec(block, ix)


def _const(block):
    return pl.BlockSpec(block, lambda p, j: (0, 0))


_PARAM_SPECS = [
    _const((D, H2)),
    _const((1, H2)),
    _const((1, H2)),
    _const((1, H2)),
    _const((H2, D)),
    _const((1, D)),
    _const((1, D)),
    _const((1, D)),
]

_DENSE_SCRATCH = [
    pltpu.VMEM((N, H2), jnp.float32),
    pltpu.VMEM((N, D), jnp.float32),
    pltpu.VMEM((2, H2), jnp.float32),
    pltpu.VMEM((2, D), jnp.float32),
]


def _dense_layer(a0, a1, params):
    return pl.pallas_call(
        _layer_body,
        grid=(3, G),
        in_specs=[_gated((BLK, DH), 0), _gated((BLK, DH), 0)] + _PARAM_SPECS,
        out_specs=(_gated((BLK, DH), 2), _gated((BLK, DH), 2)),
        out_shape=(jax.ShapeDtypeStruct((N, DH), jnp.float32),
                   jax.ShapeDtypeStruct((N, DH), jnp.float32)),
        scratch_shapes=_DENSE_SCRATCH,
        compiler_params=pltpu.CompilerParams(
            dimension_semantics=("arbitrary", "arbitrary")),
    )(a0, a1, *params)


def _dense_readout(a0, a1, params, batch_col, wc, bc):
    return pl.pallas_call(
        _readout_body,
        grid=(3, G),
        in_specs=[_gated((BLK, DH), 0), _gated((BLK, DH), 0)] + _PARAM_SPECS
        + [_gated((BLK, 1), 2), _const((D, 1)), _const((1, 1))],
        out_specs=pl.BlockSpec((B, 1), lambda p, j: (0, 0)),
        out_shape=jax.ShapeDtypeStruct((B, 1), jnp.float32),
        scratch_shapes=_DENSE_SCRATCH + [pltpu.VMEM((B, D), jnp.float32),
                                         pltpu.VMEM((B, 128), jnp.float32)],
        compiler_params=pltpu.CompilerParams(
            dimension_semantics=("arbitrary", "arbitrary")),
    )(a0, a1, *params, batch_col, wc, bc)


def kernel(x, edge_index, batch, batch_size, W1, b1, g1, be1, W2, b2, g2, be2,
           Wc, bc):
    src_r = edge_index[0].reshape(NS, NB, CPB, CHUNK)
    dst_r = edge_index[1].reshape(NS, NB, CPB, CHUNK)
    batch_col = batch.reshape(N, 1)
    h0 = x[:, :DH]
    h1 = x[:, DH:]
    out = None
    for i in range(3):
        params = (W1[i], b1[i].reshape(1, H2), g1[i].reshape(1, H2),
                  be1[i].reshape(1, H2), W2[i], b2[i].reshape(1, D),
                  g2[i].reshape(1, D), be2[i].reshape(1, D))
        a0, a1 = _sc_scatter(h0, h1, src_r, dst_r)
        if i < 2:
            h0, h1 = _dense_layer(a0, a1, params)
        else:
            out = _dense_readout(a0, a1, params, batch_col, Wc,
                                 bc.reshape(1, 1))
    return out
